# baseline (speedup 1.0000x reference)
import functools

import jax
import jax.numpy as jnp
from jax import lax
from jax.experimental import pallas as pl
from jax.experimental.pallas import tpu as pltpu
from jax.experimental.pallas import tpu_sc as plsc

SEQ_BLOCK = 1024


def _add_kernel(in_ref, pos_ref, out_ref):
    out_ref[...] = in_ref[...] + pos_ref[...]


def _tc_kernel(input_embeddings, pos_table):
    batch, seq_len, dim = input_embeddings.shape
    s_blocks = seq_len // SEQ_BLOCK
    grid = (s_blocks, batch)
    return pl.pallas_call(
        _add_kernel,
        grid=grid,
        in_specs=[
            pl.BlockSpec((1, SEQ_BLOCK, dim), lambda s, b: (b, s, 0)),
            pl.BlockSpec((SEQ_BLOCK, dim), lambda s, b: (s, 0)),
        ],
        out_specs=pl.BlockSpec((1, SEQ_BLOCK, dim), lambda s, b: (b, s, 0)),
        out_shape=jax.ShapeDtypeStruct((batch, seq_len, dim), input_embeddings.dtype),
        compiler_params=pltpu.CompilerParams(
            dimension_semantics=("parallel", "parallel"),
        ),
    )(input_embeddings, pos_table)


_NC = 2
_NS = 16
_NW = _NC * _NS
_LANES = 16
_R = 16


def _sc_body(in_hbm, pos_hbm, out_hbm, in_v, pos_v):
    wid = lax.axis_index("s") * _NC + lax.axis_index("c")
    seq_per_w = 2048 // _NW
    chunk = _R * 2048
    n_vec = chunk // _LANES

    def add_body(i, _):
        sl = pl.ds(i * _LANES, _LANES)
        in_v[sl] = in_v[sl] + pos_v[sl]
        return 0

    for sc_i in range(seq_per_w // _R):
        row = (wid * seq_per_w + sc_i * _R) * 2048
        pltpu.sync_copy(pos_hbm.at[pl.ds(row, chunk)], pos_v)
        for b in range(4):
            off = b * (2048 * 2048) + row
            pltpu.sync_copy(in_hbm.at[pl.ds(off, chunk)], in_v)
            lax.fori_loop(0, n_vec, add_body, 0)
            pltpu.sync_copy(in_v, out_hbm.at[pl.ds(off, chunk)])


def _sc_kernel(input_embeddings, pos_table):
    batch, seq_len, dim = input_embeddings.shape
    n = batch * seq_len * dim
    mesh = plsc.VectorSubcoreMesh(core_axis_name="c", subcore_axis_name="s")
    k = pl.kernel(
        _sc_body,
        out_type=jax.ShapeDtypeStruct((n,), jnp.float32),
        mesh=mesh,
        scratch_types=[
            pltpu.VMEM((_R * 2048,), jnp.float32),
            pltpu.VMEM((_R * 2048,), jnp.float32),
        ],
    )
    out = k(input_embeddings.reshape(n), pos_table.reshape(seq_len * dim))
    return out.reshape(batch, seq_len, dim)


def kernel(input_embeddings, pos_table):
    return _sc_kernel(input_embeddings, pos_table)


# speedup vs baseline: 1.7143x; 1.7143x over previous
import functools

import jax
import jax.numpy as jnp
from jax import lax
from jax.experimental import pallas as pl
from jax.experimental.pallas import tpu as pltpu
from jax.experimental.pallas import tpu_sc as plsc

SEQ_BLOCK = 1024


def _add_kernel(in_ref, pos_ref, out_ref):
    out_ref[...] = in_ref[...] + pos_ref[...]


def _tc_kernel(input_embeddings, pos_table):
    batch, seq_len, dim = input_embeddings.shape
    s_blocks = seq_len // SEQ_BLOCK
    grid = (s_blocks, batch)
    return pl.pallas_call(
        _add_kernel,
        grid=grid,
        in_specs=[
            pl.BlockSpec((1, SEQ_BLOCK, dim), lambda s, b: (b, s, 0)),
            pl.BlockSpec((SEQ_BLOCK, dim), lambda s, b: (s, 0)),
        ],
        out_specs=pl.BlockSpec((1, SEQ_BLOCK, dim), lambda s, b: (b, s, 0)),
        out_shape=jax.ShapeDtypeStruct((batch, seq_len, dim), input_embeddings.dtype),
        compiler_params=pltpu.CompilerParams(
            dimension_semantics=("parallel", "parallel"),
        ),
    )(input_embeddings, pos_table)


_NC = 2
_NS = 16
_NW = _NC * _NS
_LANES = 16
_R = 8
_BATCH = 4
_SEQ = 2048
_DIM = 2048
_CHUNK = _R * _DIM
_UNROLL = 8


def _sc_body(in_hbm, pos_hbm, out_hbm,
             in_v0, in_v1, in_v2, pos_v0, pos_v1,
             si0, si1, si2, so0, so1, so2, sp0, sp1):
    in_bufs = [in_v0, in_v1, in_v2]
    in_sems = [si0, si1, si2]
    out_sems = [so0, so1, so2]
    pos_bufs = [pos_v0, pos_v1]
    pos_sems = [sp0, sp1]

    wid = lax.axis_index("s") * _NC + lax.axis_index("c")
    seq_per_w = _SEQ // _NW
    n_chunks = seq_per_w // _R
    n_steps = n_chunks * _BATCH
    base_row = wid * seq_per_w

    def in_off(step):
        sc_i, b = step // _BATCH, step % _BATCH
        return b * (_SEQ * _DIM) + (base_row + sc_i * _R) * _DIM

    def start_in(step):
        return pltpu.async_copy(
            in_hbm.at[pl.ds(in_off(step), _CHUNK)],
            in_bufs[step % 3], in_sems[step % 3])

    def start_out(step):
        return pltpu.async_copy(
            in_bufs[step % 3],
            out_hbm.at[pl.ds(in_off(step), _CHUNK)], out_sems[step % 3])

    def start_pos(sc_i):
        return pltpu.async_copy(
            pos_hbm.at[pl.ds((base_row + sc_i * _R) * _DIM, _CHUNK)],
            pos_bufs[sc_i % 2], pos_sems[sc_i % 2])

    n_vec = _CHUNK // _LANES

    def make_add(buf, pos_buf):
        def add_body(i, _):
            for u in range(_UNROLL):
                sl = pl.ds((i * _UNROLL + u) * _LANES, _LANES)
                plsc.addupdate(buf.at[sl], pos_buf[sl])
            return 0
        return add_body

    in_h = {}
    out_h = {}
    pos_h = {}
    pos_h[0] = start_pos(0)
    in_h[0] = start_in(0)
    for s in range(n_steps):
        sc_i, b = s // _BATCH, s % _BATCH
        if s + 1 < n_steps:
            if s - 2 >= 0:
                out_h[s - 2].wait()
            in_h[s + 1] = start_in(s + 1)
        if b == 0 and sc_i + 1 < n_chunks:
            pos_h[sc_i + 1] = start_pos(sc_i + 1)
        in_h[s].wait()
        if b == 0:
            pos_h[sc_i].wait()
        lax.fori_loop(0, n_vec // _UNROLL,
                      make_add(in_bufs[s % 3], pos_bufs[sc_i % 2]), 0)
        out_h[s] = start_out(s)
    for s in (n_steps - 3, n_steps - 2, n_steps - 1):
        out_h[s].wait()


def _sc_kernel(input_embeddings, pos_table):
    batch, seq_len, dim = input_embeddings.shape
    n = batch * seq_len * dim
    mesh = plsc.VectorSubcoreMesh(core_axis_name="c", subcore_axis_name="s")
    k = pl.kernel(
        _sc_body,
        out_type=jax.ShapeDtypeStruct((n,), jnp.float32),
        mesh=mesh,
        scratch_types=(
            [pltpu.VMEM((_CHUNK,), jnp.float32)] * 3
            + [pltpu.VMEM((_CHUNK,), jnp.float32)] * 2
            + [pltpu.SemaphoreType.DMA] * 8
        ),
    )
    out = k(input_embeddings.reshape(n), pos_table.reshape(seq_len * dim))
    return out.reshape(batch, seq_len, dim)


def kernel(input_embeddings, pos_table):
    return _sc_kernel(input_embeddings, pos_table)


# speedup vs baseline: 2.0121x; 1.1737x over previous
import functools

import jax
import jax.numpy as jnp
from jax import lax
from jax.experimental import pallas as pl
from jax.experimental.pallas import tpu as pltpu
from jax.experimental.pallas import tpu_sc as plsc

SEQ_BLOCK = 1024


def _add_kernel(in_ref, pos_ref, out_ref):
    out_ref[...] = in_ref[...] + pos_ref[...]


def _tc_kernel(input_embeddings, pos_table):
    batch, seq_len, dim = input_embeddings.shape
    s_blocks = seq_len // SEQ_BLOCK
    grid = (s_blocks, batch)
    return pl.pallas_call(
        _add_kernel,
        grid=grid,
        in_specs=[
            pl.BlockSpec((1, SEQ_BLOCK, dim), lambda s, b: (b, s, 0)),
            pl.BlockSpec((SEQ_BLOCK, dim), lambda s, b: (s, 0)),
        ],
        out_specs=pl.BlockSpec((1, SEQ_BLOCK, dim), lambda s, b: (b, s, 0)),
        out_shape=jax.ShapeDtypeStruct((batch, seq_len, dim), input_embeddings.dtype),
        compiler_params=pltpu.CompilerParams(
            dimension_semantics=("parallel", "parallel"),
        ),
    )(input_embeddings, pos_table)


_NC = 2
_NS = 16
_NW = _NC * _NS
_LANES = 16
_R = 8
_BATCH = 4
_SEQ = 2048
_DIM = 2048
_CHUNK = _R * _DIM
_UNROLL = 8


def _sc_body(in_hbm, pos_hbm, out_hbm,
             in_v0, in_v1, in_v2, pos_v0, pos_v1,
             si0, si1, si2, so0, so1, so2, sp0, sp1):
    in_bufs = [in_v0, in_v1, in_v2]
    in_sems = [si0, si1, si2]
    out_sems = [so0, so1, so2]
    pos_bufs = [pos_v0, pos_v1]
    pos_sems = [sp0, sp1]

    wid = lax.axis_index("s") * _NC + lax.axis_index("c")
    seq_per_w = _SEQ // _NW
    n_chunks = seq_per_w // _R
    n_steps = n_chunks * _BATCH
    base_row = wid * seq_per_w

    def in_off(step):
        sc_i, b = step // _BATCH, step % _BATCH
        return b * (_SEQ * _DIM) + (base_row + sc_i * _R) * _DIM

    def start_in(step):
        return pltpu.async_copy(
            in_hbm.at[pl.ds(in_off(step), _CHUNK)],
            in_bufs[step % 3], in_sems[step % 3])

    def start_out(step):
        return pltpu.async_copy(
            in_bufs[step % 3],
            out_hbm.at[pl.ds(in_off(step), _CHUNK)], out_sems[step % 3])

    def start_pos(sc_i):
        return pltpu.async_copy(
            pos_hbm.at[pl.ds((base_row + sc_i * _R) * _DIM, _CHUNK)],
            pos_bufs[sc_i % 2], pos_sems[sc_i % 2])

    n_vec = _CHUNK // _LANES

    def make_add(buf, pos_buf):
        def add_body(i, _):
            for u in range(_UNROLL):
                sl = pl.ds((i * _UNROLL + u) * _LANES, _LANES)
                plsc.addupdate(buf.at[sl], pos_buf[sl])
            return 0
        return add_body

    in_h = {}
    out_h = {}
    pos_h = {}
    pos_h[0] = start_pos(0)
    in_h[0] = start_in(0)
    for s in range(n_steps):
        sc_i, b = s // _BATCH, s % _BATCH
        if s + 1 < n_steps:
            if s - 2 >= 0:
                out_h[s - 2].wait()
            in_h[s + 1] = start_in(s + 1)
        if b == 0 and sc_i + 1 < n_chunks:
            pos_h[sc_i + 1] = start_pos(sc_i + 1)
        in_h[s].wait()
        if b == 0:
            pos_h[sc_i].wait()
        lax.fori_loop(0, n_vec // _UNROLL,
                      make_add(in_bufs[s % 3], pos_bufs[sc_i % 2]), 0)
        out_h[s] = start_out(s)
    for s in (n_steps - 3, n_steps - 2, n_steps - 1):
        out_h[s].wait()


def _sc_kernel(input_embeddings, pos_table):
    batch, seq_len, dim = input_embeddings.shape
    n = batch * seq_len * dim
    mesh = plsc.VectorSubcoreMesh(core_axis_name="c", subcore_axis_name="s")
    k = pl.kernel(
        _sc_body,
        out_type=jax.ShapeDtypeStruct((n,), jnp.float32),
        mesh=mesh,
        scratch_types=(
            [pltpu.VMEM((_CHUNK,), jnp.float32)] * 3
            + [pltpu.VMEM((_CHUNK,), jnp.float32)] * 2
            + [pltpu.SemaphoreType.DMA] * 8
        ),
    )
    out = k(input_embeddings.reshape(n), pos_table.reshape(seq_len * dim))
    return out.reshape(batch, seq_len, dim)


_SPLIT = 1792
_TC_SEQ_BLOCK = 896


def _sc_tail_body(in_hbm, pos_hbm, out_hbm,
                  in_v0, in_v1, in_v2, pos_v0,
                  si0, si1, si2, so0, so1, so2, sp0):
    in_bufs = [in_v0, in_v1, in_v2]
    in_sems = [si0, si1, si2]
    out_sems = [so0, so1, so2]
    tail = _SEQ - _SPLIT
    rows_per_w = tail // _NW
    wid = lax.axis_index("s") * _NC + lax.axis_index("c")
    row = _SPLIT + wid * rows_per_w
    chunk = rows_per_w * _DIM

    def start_in(b):
        return pltpu.async_copy(
            in_hbm.at[pl.ds(b * (_SEQ * _DIM) + row * _DIM, chunk)],
            in_bufs[b % 3], in_sems[b % 3])

    def start_out(b):
        return pltpu.async_copy(
            in_bufs[b % 3],
            out_hbm.at[pl.ds(b * (tail * _DIM) + wid * chunk, chunk)],
            out_sems[b % 3])

    n_vec = chunk // _LANES

    def make_add(buf):
        def add_body(i, _):
            for u in range(_UNROLL):
                sl = pl.ds((i * _UNROLL + u) * _LANES, _LANES)
                plsc.addupdate(buf.at[sl], pos_v0[sl])
            return 0
        return add_body

    pos_h = pltpu.async_copy(
        pos_hbm.at[pl.ds(row * _DIM, chunk)], pos_v0, sp0)
    in_h = {0: start_in(0)}
    out_h = {}
    pos_h.wait()
    for b in range(_BATCH):
        if b + 1 < _BATCH:
            in_h[b + 1] = start_in(b + 1)
        in_h[b].wait()
        lax.fori_loop(0, n_vec // _UNROLL, make_add(in_bufs[b % 3]), 0)
        out_h[b] = start_out(b)
    for b in range(1, _BATCH):
        out_h[b].wait()
    out_h[0].wait()


def _hybrid_kernel(input_embeddings, pos_table):
    batch, seq_len, dim = input_embeddings.shape
    n = batch * seq_len * dim
    tail = seq_len - _SPLIT

    mesh = plsc.VectorSubcoreMesh(core_axis_name="c", subcore_axis_name="s")
    sc_k = pl.kernel(
        _sc_tail_body,
        out_type=jax.ShapeDtypeStruct((batch * tail * dim,), jnp.float32),
        mesh=mesh,
        scratch_types=(
            [pltpu.VMEM(((tail // _NW) * dim,), jnp.float32)] * 4
            + [pltpu.SemaphoreType.DMA] * 7
        ),
    )
    out_sc = sc_k(input_embeddings.reshape(n), pos_table.reshape(seq_len * dim))
    out_sc = out_sc.reshape(batch, tail, dim)

    s_blocks = _SPLIT // _TC_SEQ_BLOCK
    out_tc = pl.pallas_call(
        _add_kernel,
        grid=(s_blocks, batch),
        in_specs=[
            pl.BlockSpec((1, _TC_SEQ_BLOCK, dim), lambda s, b: (b, s, 0)),
            pl.BlockSpec((_TC_SEQ_BLOCK, dim), lambda s, b: (s, 0)),
        ],
        out_specs=pl.BlockSpec((1, _TC_SEQ_BLOCK, dim), lambda s, b: (b, s, 0)),
        out_shape=jax.ShapeDtypeStruct((batch, _SPLIT, dim), input_embeddings.dtype),
        compiler_params=pltpu.CompilerParams(
            dimension_semantics=("parallel", "parallel"),
        ),
    )(input_embeddings, pos_table)

    return jnp.concatenate([out_tc, out_sc], axis=1)


def kernel(input_embeddings, pos_table):
    return _hybrid_kernel(input_embeddings, pos_table)


# speedup vs baseline: 7.8855x; 3.9190x over previous
import functools

import jax
import jax.numpy as jnp
from jax import lax
from jax.experimental import pallas as pl
from jax.experimental.pallas import tpu as pltpu
from jax.experimental.pallas import tpu_sc as plsc

SEQ_BLOCK = 1024


def _add_kernel(in_ref, pos_ref, out_ref):
    out_ref[...] = in_ref[...] + pos_ref[...]


def _tc_kernel(input_embeddings, pos_table):
    batch, seq_len, dim = input_embeddings.shape
    s_blocks = seq_len // SEQ_BLOCK
    grid = (s_blocks, batch)
    return pl.pallas_call(
        _add_kernel,
        grid=grid,
        in_specs=[
            pl.BlockSpec((1, SEQ_BLOCK, dim), lambda s, b: (b, s, 0)),
            pl.BlockSpec((SEQ_BLOCK, dim), lambda s, b: (s, 0)),
        ],
        out_specs=pl.BlockSpec((1, SEQ_BLOCK, dim), lambda s, b: (b, s, 0)),
        out_shape=jax.ShapeDtypeStruct((batch, seq_len, dim), input_embeddings.dtype),
        compiler_params=pltpu.CompilerParams(
            dimension_semantics=("parallel", "parallel"),
        ),
    )(input_embeddings, pos_table)


_NC = 2
_NS = 16
_NW = _NC * _NS
_LANES = 16
_R = 8
_BATCH = 4
_SEQ = 2048
_DIM = 2048
_CHUNK = _R * _DIM
_UNROLL = 8


def _sc_body(in_hbm, pos_hbm, out_hbm,
             in_v0, in_v1, in_v2, pos_v0, pos_v1,
             si0, si1, si2, so0, so1, so2, sp0, sp1):
    in_bufs = [in_v0, in_v1, in_v2]
    in_sems = [si0, si1, si2]
    out_sems = [so0, so1, so2]
    pos_bufs = [pos_v0, pos_v1]
    pos_sems = [sp0, sp1]

    wid = lax.axis_index("s") * _NC + lax.axis_index("c")
    seq_per_w = _SEQ // _NW
    n_chunks = seq_per_w // _R
    n_steps = n_chunks * _BATCH
    base_row = wid * seq_per_w

    def in_off(step):
        sc_i, b = step // _BATCH, step % _BATCH
        return b * (_SEQ * _DIM) + (base_row + sc_i * _R) * _DIM

    def start_in(step):
        return pltpu.async_copy(
            in_hbm.at[pl.ds(in_off(step), _CHUNK)],
            in_bufs[step % 3], in_sems[step % 3])

    def start_out(step):
        return pltpu.async_copy(
            in_bufs[step % 3],
            out_hbm.at[pl.ds(in_off(step), _CHUNK)], out_sems[step % 3])

    def start_pos(sc_i):
        return pltpu.async_copy(
            pos_hbm.at[pl.ds((base_row + sc_i * _R) * _DIM, _CHUNK)],
            pos_bufs[sc_i % 2], pos_sems[sc_i % 2])

    n_vec = _CHUNK // _LANES

    def make_add(buf, pos_buf):
        def add_body(i, _):
            for u in range(_UNROLL):
                sl = pl.ds((i * _UNROLL + u) * _LANES, _LANES)
                plsc.addupdate(buf.at[sl], pos_buf[sl])
            return 0
        return add_body

    in_h = {}
    out_h = {}
    pos_h = {}
    pos_h[0] = start_pos(0)
    in_h[0] = start_in(0)
    for s in range(n_steps):
        sc_i, b = s // _BATCH, s % _BATCH
        if s + 1 < n_steps:
            if s - 2 >= 0:
                out_h[s - 2].wait()
            in_h[s + 1] = start_in(s + 1)
        if b == 0 and sc_i + 1 < n_chunks:
            pos_h[sc_i + 1] = start_pos(sc_i + 1)
        in_h[s].wait()
        if b == 0:
            pos_h[sc_i].wait()
        lax.fori_loop(0, n_vec // _UNROLL,
                      make_add(in_bufs[s % 3], pos_bufs[sc_i % 2]), 0)
        out_h[s] = start_out(s)
    for s in (n_steps - 3, n_steps - 2, n_steps - 1):
        out_h[s].wait()


def _sc_kernel(input_embeddings, pos_table):
    batch, seq_len, dim = input_embeddings.shape
    n = batch * seq_len * dim
    mesh = plsc.VectorSubcoreMesh(core_axis_name="c", subcore_axis_name="s")
    k = pl.kernel(
        _sc_body,
        out_type=jax.ShapeDtypeStruct((n,), jnp.float32),
        mesh=mesh,
        scratch_types=(
            [pltpu.VMEM((_CHUNK,), jnp.float32)] * 3
            + [pltpu.VMEM((_CHUNK,), jnp.float32)] * 2
            + [pltpu.SemaphoreType.DMA] * 8
        ),
    )
    out = k(input_embeddings.reshape(n), pos_table.reshape(seq_len * dim))
    return out.reshape(batch, seq_len, dim)


_SPLIT = 1792
_TC_SEQ_BLOCK = 896


def _sc_tail_body(in_hbm, pos_hbm, out_hbm,
                  in_v0, in_v1, in_v2, pos_v0,
                  si0, si1, si2, so0, so1, so2, sp0):
    in_bufs = [in_v0, in_v1, in_v2]
    in_sems = [si0, si1, si2]
    out_sems = [so0, so1, so2]
    tail = _SEQ - _SPLIT
    rows_per_w = tail // _NW
    wid = lax.axis_index("s") * _NC + lax.axis_index("c")
    row = _SPLIT + wid * rows_per_w
    chunk = rows_per_w * _DIM

    def start_in(b):
        return pltpu.async_copy(
            in_hbm.at[pl.ds(b * (_SEQ * _DIM) + row * _DIM, chunk)],
            in_bufs[b % 3], in_sems[b % 3])

    def start_out(b):
        return pltpu.async_copy(
            in_bufs[b % 3],
            out_hbm.at[pl.ds(b * (tail * _DIM) + wid * chunk, chunk)],
            out_sems[b % 3])

    n_vec = chunk // _LANES

    def make_add(buf):
        def add_body(i, _):
            for u in range(_UNROLL):
                sl = pl.ds((i * _UNROLL + u) * _LANES, _LANES)
                plsc.addupdate(buf.at[sl], pos_v0[sl])
            return 0
        return add_body

    pos_h = pltpu.async_copy(
        pos_hbm.at[pl.ds(row * _DIM, chunk)], pos_v0, sp0)
    in_h = {0: start_in(0)}
    out_h = {}
    pos_h.wait()
    for b in range(_BATCH):
        if b + 1 < _BATCH:
            in_h[b + 1] = start_in(b + 1)
        in_h[b].wait()
        lax.fori_loop(0, n_vec // _UNROLL, make_add(in_bufs[b % 3]), 0)
        out_h[b] = start_out(b)
    for b in range(1, _BATCH):
        out_h[b].wait()
    out_h[0].wait()


def _hybrid_kernel(input_embeddings, pos_table):
    batch, seq_len, dim = input_embeddings.shape
    n = batch * seq_len * dim
    tail = seq_len - _SPLIT

    mesh = plsc.VectorSubcoreMesh(core_axis_name="c", subcore_axis_name="s")
    sc_k = pl.kernel(
        _sc_tail_body,
        out_type=jax.ShapeDtypeStruct((batch * tail * dim,), jnp.float32),
        mesh=mesh,
        scratch_types=(
            [pltpu.VMEM(((tail // _NW) * dim,), jnp.float32)] * 4
            + [pltpu.SemaphoreType.DMA] * 7
        ),
    )
    out_sc = sc_k(input_embeddings.reshape(n), pos_table.reshape(seq_len * dim))
    out_sc = out_sc.reshape(batch, tail, dim)

    s_blocks = _SPLIT // _TC_SEQ_BLOCK
    out_tc = pl.pallas_call(
        _add_kernel,
        grid=(s_blocks, batch),
        in_specs=[
            pl.BlockSpec((1, _TC_SEQ_BLOCK, dim), lambda s, b: (b, s, 0)),
            pl.BlockSpec((_TC_SEQ_BLOCK, dim), lambda s, b: (s, 0)),
        ],
        out_specs=pl.BlockSpec((1, _TC_SEQ_BLOCK, dim), lambda s, b: (b, s, 0)),
        out_shape=jax.ShapeDtypeStruct((batch, _SPLIT, dim), input_embeddings.dtype),
        compiler_params=pltpu.CompilerParams(
            dimension_semantics=("parallel", "parallel"),
        ),
    )(input_embeddings, pos_table)

    return jnp.concatenate([out_tc, out_sc], axis=1)


_ROW_BLOCK = 1024


def _flat_add_kernel(in_ref, pos_ref, out_ref):
    i = pl.program_id(0)
    sl = pl.ds((i % 2) * _ROW_BLOCK, _ROW_BLOCK)
    out_ref[...] = in_ref[...] + pos_ref[sl, :]


def _tc_flat_kernel(input_embeddings, pos_table):
    batch, seq_len, dim = input_embeddings.shape
    rows = batch * seq_len
    flat = input_embeddings.reshape(rows, dim)
    out = pl.pallas_call(
        _flat_add_kernel,
        grid=(rows // _ROW_BLOCK,),
        in_specs=[
            pl.BlockSpec((_ROW_BLOCK, dim), lambda i: (i, 0)),
            pl.BlockSpec((seq_len, dim), lambda i: (0, 0)),
        ],
        out_specs=pl.BlockSpec((_ROW_BLOCK, dim), lambda i: (i, 0)),
        out_shape=jax.ShapeDtypeStruct((rows, dim), input_embeddings.dtype),
        compiler_params=pltpu.CompilerParams(
            dimension_semantics=("arbitrary",),
        ),
    )(flat, pos_table)
    return out.reshape(batch, seq_len, dim)


def kernel(input_embeddings, pos_table):
    return _tc_flat_kernel(input_embeddings, pos_table)
